# baseline (speedup 1.0000x reference)
import functools

import jax
import jax.numpy as jnp
from jax import lax
from jax.experimental import pallas as pl
from jax.experimental.pallas import tpu as pltpu
from jax.experimental.pallas import tpu_sc as plsc

N = 10000
E = 320000
D = 128
NC, NS = 2, 16
NW = NC * NS
EPW = E // NW
CH = 80
NCHUNK = EPW // CH
NBLK = 15
NBLKS = NCHUNK // NBLK
NCP = (NBLKS + 1) * NBLK
NP = 10240
RPS = NP // NS
L = 16

BN = 1024


def _d1_body(x_ref, w_ref, b_ref, out_ref):
    h = jnp.dot(x_ref[...], w_ref[...], preferred_element_type=jnp.float32)
    out_ref[...] = jnp.maximum(h + b_ref[...], 0.0)


def _d1(x, w, b):
    return pl.pallas_call(
        _d1_body,
        grid=(NP // BN,),
        in_specs=[
            pl.BlockSpec((BN, D), lambda i: (i, 0)),
            pl.BlockSpec((D, D), lambda i: (0, 0)),
            pl.BlockSpec((1, D), lambda i: (0, 0)),
        ],
        out_specs=pl.BlockSpec((BN, D), lambda i: (i, 0)),
        out_shape=jax.ShapeDtypeStruct((N, D), jnp.float32),
    )(x, w, b)


_MESH = plsc.VectorSubcoreMesh(
    core_axis_name="c", subcore_axis_name="s", num_cores=NC, num_subcores=NS)


@functools.partial(
    pl.kernel,
    out_type=(jax.ShapeDtypeStruct((NC, NP, D), jnp.float32),
              jax.ShapeDtypeStruct((NW, N), jnp.float32)),
    mesh=_MESH,
    compiler_params=pltpu.CompilerParams(needs_layout_passes=False),
    scratch_types=[
        pltpu.VMEM((NBLK, 2, CH), jnp.int32),
        pltpu.VMEM((NBLK, 2, CH), jnp.int32),
        pltpu.VMEM((N,), jnp.float32),
        pltpu.VMEM((CH, D), jnp.float32),
        pltpu.VMEM((CH, D), jnp.float32),
        pltpu.VMEM((CH, D), jnp.float32),
        pltpu.VMEM_SHARED((NP, D), jnp.float32),
        pltpu.SemaphoreType.DMA,
        pltpu.SemaphoreType.DMA,
        pltpu.SemaphoreType.DMA,
        pltpu.SemaphoreType.DMA,
        pltpu.SemaphoreType.DMA,
        pltpu.SemaphoreType.DMA,
        pltpu.SemaphoreType.DMA,
        pltpu.SemaphoreType.DMA,
    ],
)
def _sc_agg(h_hbm, idx_hbm, out_hbm, deg_hbm,
            tib_a, tib_b, hist_v, r0, r1, r2, agg_sh,
            g0, g1, g2, s0, s1, s2, isem_a, isem_b):
    cid = lax.axis_index("c")
    sid = lax.axis_index("s")
    wid = cid * NS + sid
    rbufs = (r0, r1, r2)
    gsems = (g0, g1, g2)

    def _blk_load(b, tib, isem):
        pltpu.async_copy(idx_hbm.at[wid, pl.ds(NBLK * b, NBLK)], tib, isem)

    def _blk_wait(tib, isem):
        pltpu.make_async_copy(idx_hbm.at[wid, pl.ds(0, NBLK)], tib,
                              isem).wait()

    def _gather(idx_row, buf, sem):
        pltpu.async_copy(h_hbm.at[idx_row], buf, sem)

    def _gwait(buf, sem):
        pltpu.make_async_copy(h_hbm.at[tib_a.at[0, 0]], buf, sem).wait()

    ssems = (s0, s1, s2)

    def _ascat(idx_row, k):
        pltpu.async_copy(rbufs[k], agg_sh.at[idx_row], ssems[k], add=True)

    def _swait(k):
        pltpu.make_async_copy(rbufs[k], agg_sh.at[tib_a.at[0, 1]],
                              ssems[k]).wait()

    ones = jnp.ones((L,), jnp.float32)

    def _hist(tib, k):
        for j in range(CH // L):
            d16 = tib[k, 1, pl.ds(j * L, L)]
            plsc.addupdate_scatter(hist_v, [d16], ones)

    _blk_load(0, tib_a, isem_a)
    zeros16 = jnp.zeros((L,), jnp.float32)

    def _zero_hist(j, carry):
        hist_v[pl.ds(j * L, L)] = zeros16
        return carry

    lax.fori_loop(0, N // L, _zero_hist, 0)
    _blk_wait(tib_a, isem_a)
    _gather(tib_a.at[1, 0], rbufs[1], gsems[1])

    def _zero_r0(j, carry):
        r0[j % CH, pl.ds((j // CH) * L, L)] = zeros16
        return carry

    lax.fori_loop(0, CH * (D // L), _zero_r0, 0)
    for i in range(RPS // CH):
        pltpu.sync_copy(r0, agg_sh.at[pl.ds(sid * RPS + i * CH, CH)])
    _gather(tib_a.at[0, 0], rbufs[0], gsems[0])
    plsc.subcore_barrier()

    def _half(b, cur, oth, isem_oth, first):
        for m in range(NBLK):
            k = m % 3
            kg = (m + 2) % 3
            _gwait(rbufs[k], gsems[k])
            _ascat(cur.at[m, 1], k)
            _hist(cur, m)
            if not (first and m == 0):
                _swait(kg)
            if m == 1:
                _blk_load(b + 1, oth, isem_oth)
            if m == 12:
                _blk_wait(oth, isem_oth)
            if m < NBLK - 2:
                _gather(cur.at[m + 2, 0], rbufs[kg], gsems[kg])
            else:
                _gather(oth.at[m + 2 - NBLK, 0], rbufs[kg], gsems[kg])

    _half(0, tib_a, tib_b, isem_b, True)

    def _pair(i, carry):
        b = 2 * i + 1
        _half(b, tib_b, tib_a, isem_a, False)
        _half(b + 1, tib_a, tib_b, isem_b, False)
        return carry

    lax.fori_loop(0, (NBLKS - 2) // 2, _pair, 0)
    _half(NBLKS - 1, tib_b, tib_a, isem_a, False)
    for m in range(5):
        k = m % 3
        kg = (m + 2) % 3
        _gwait(rbufs[k], gsems[k])
        _ascat(tib_a.at[m, 1], k)
        _hist(tib_a, m)
        _swait(kg)
        if m + 2 < 5:
            _gather(tib_a.at[m + 2, 0], rbufs[kg], gsems[kg])
    _swait(4 % 3)

    pltpu.sync_copy(hist_v, deg_hbm.at[wid])
    plsc.subcore_barrier()
    pltpu.sync_copy(agg_sh.at[pl.ds(sid * RPS, RPS)],
                    out_hbm.at[cid, pl.ds(sid * RPS, RPS)])


def _out_body(part_ref, deg_ref, h_ref, ws_ref, bs_ref, wn_ref, w2_ref,
              b2_ref, o_ref):
    agg = part_ref[0] + part_ref[1]
    degt = jnp.transpose(deg_ref[...])
    deg = jnp.maximum(jnp.sum(degt, axis=1, keepdims=True), 1.0)
    h_neigh = agg / deg
    h = h_ref[...]
    h2 = jnp.dot(h, ws_ref[...], preferred_element_type=jnp.float32)
    h2 = h2 + jnp.dot(h_neigh, wn_ref[...], preferred_element_type=jnp.float32)
    h2 = jnp.maximum(h2 + bs_ref[...], 0.0)
    o = jnp.dot(h2, w2_ref[...], preferred_element_type=jnp.float32)
    o_ref[...] = jnp.maximum(o + b2_ref[...], 0.0)


def _out(part, deg, h, ws, bs, wn, w2, b2):
    return pl.pallas_call(
        _out_body,
        grid=(NP // BN,),
        in_specs=[
            pl.BlockSpec((NC, BN, D), lambda i: (0, i, 0)),
            pl.BlockSpec((NW, BN), lambda i: (0, i)),
            pl.BlockSpec((BN, D), lambda i: (i, 0)),
            pl.BlockSpec((D, D), lambda i: (0, 0)),
            pl.BlockSpec((1, D), lambda i: (0, 0)),
            pl.BlockSpec((D, D), lambda i: (0, 0)),
            pl.BlockSpec((D, D), lambda i: (0, 0)),
            pl.BlockSpec((1, D), lambda i: (0, 0)),
        ],
        out_specs=pl.BlockSpec((BN, D), lambda i: (i, 0)),
        out_shape=jax.ShapeDtypeStruct((N, D), jnp.float32),
    )(part, deg, h, ws, bs, wn, w2, b2)


def kernel(x, edge_index, W_d1, b_d1, W_self, b_self, W_neigh, W_d2, b_d2):
    h = _d1(x, W_d1, b_d1.reshape(1, D))
    idx = jnp.transpose(edge_index.reshape(2, NW, NCHUNK, CH), (1, 2, 0, 3))
    idx = jnp.pad(idx, ((0, 0), (0, NCP - NCHUNK), (0, 0), (0, 0)))
    part, deg = _sc_agg(h, idx)
    return _out(part, deg, h, W_self,
                b_self.reshape(1, D), W_neigh, W_d2, b_d2.reshape(1, D))


# speedup vs baseline: 1.0085x; 1.0085x over previous
import functools

import jax
import jax.numpy as jnp
from jax import lax
from jax.experimental import pallas as pl
from jax.experimental.pallas import tpu as pltpu
from jax.experimental.pallas import tpu_sc as plsc

N = 10000
E = 320000
D = 128
NC, NS = 2, 16
NW = NC * NS
EPW = E // NW
CH = 80
NCHUNK = EPW // CH
NBLK = 15
NBLKS = NCHUNK // NBLK
NCP = (NBLKS + 1) * NBLK
NP = 10240
RPS = NP // NS
L = 16

BN = 1024


def _d1_body(x_ref, w_ref, b_ref, out_ref):
    h = jnp.dot(x_ref[...], w_ref[...], preferred_element_type=jnp.float32)
    out_ref[...] = jnp.maximum(h + b_ref[...], 0.0)


def _d1(x, w, b):
    return pl.pallas_call(
        _d1_body,
        grid=(NP // BN,),
        in_specs=[
            pl.BlockSpec((BN, D), lambda i: (i, 0)),
            pl.BlockSpec((D, D), lambda i: (0, 0)),
            pl.BlockSpec((1, D), lambda i: (0, 0)),
        ],
        out_specs=pl.BlockSpec((BN, D), lambda i: (i, 0)),
        out_shape=jax.ShapeDtypeStruct((N, D), jnp.float32),
    )(x, w, b)


_MESH = plsc.VectorSubcoreMesh(
    core_axis_name="c", subcore_axis_name="s", num_cores=NC, num_subcores=NS)


@functools.partial(
    pl.kernel,
    out_type=(jax.ShapeDtypeStruct((NC, NP, D), jnp.float32),
              jax.ShapeDtypeStruct((NW, N), jnp.float32)),
    mesh=_MESH,
    compiler_params=pltpu.CompilerParams(needs_layout_passes=False),
    scratch_types=[
        pltpu.VMEM((NBLK, 2, CH), jnp.int32),
        pltpu.VMEM((NBLK, 2, CH), jnp.int32),
        pltpu.VMEM((N,), jnp.float32),
        pltpu.VMEM((CH, D), jnp.float32),
        pltpu.VMEM((CH, D), jnp.float32),
        pltpu.VMEM((CH, D), jnp.float32),
        pltpu.VMEM_SHARED((NP, D), jnp.float32),
        pltpu.SemaphoreType.DMA,
        pltpu.SemaphoreType.DMA,
        pltpu.SemaphoreType.DMA,
        pltpu.SemaphoreType.DMA,
        pltpu.SemaphoreType.DMA,
    ],
)
def _sc_agg(h_hbm, idx_hbm, out_hbm, deg_hbm,
            tib_a, tib_b, hist_v, r0, r1, r2, agg_sh,
            g0, g1, g2, isem_a, isem_b):
    cid = lax.axis_index("c")
    sid = lax.axis_index("s")
    wid = cid * NS + sid
    rbufs = (r0, r1, r2)
    gsems = (g0, g1, g2)

    def _blk_load(b, tib, isem):
        pltpu.async_copy(idx_hbm.at[wid, pl.ds(NBLK * b, NBLK)], tib, isem)

    def _blk_wait(tib, isem):
        pltpu.make_async_copy(idx_hbm.at[wid, pl.ds(0, NBLK)], tib,
                              isem).wait()

    def _gather(idx_row, buf, sem):
        pltpu.async_copy(h_hbm.at[idx_row], buf, sem)

    def _gwait(buf, sem):
        pltpu.make_async_copy(h_hbm.at[tib_a.at[0, 0]], buf, sem).wait()

    def _scat(buf, idx_row):
        pltpu.sync_copy(buf, agg_sh.at[idx_row], add=True)

    ones = jnp.ones((L,), jnp.float32)

    def _hist(tib, k):
        for j in range(CH // L):
            d16 = tib[k, 1, pl.ds(j * L, L)]
            plsc.addupdate_scatter(hist_v, [d16], ones)

    _blk_load(0, tib_a, isem_a)
    _blk_load(1, tib_b, isem_b)
    zeros16 = jnp.zeros((L,), jnp.float32)

    def _zero_hist(j, carry):
        hist_v[pl.ds(j * L, L)] = zeros16
        return carry

    lax.fori_loop(0, N // L, _zero_hist, 0)
    _blk_wait(tib_a, isem_a)
    _gather(tib_a.at[1, 0], rbufs[1], gsems[1])
    _gather(tib_a.at[2, 0], rbufs[2], gsems[2])

    def _zero_r0(j, carry):
        r0[j % CH, pl.ds((j // CH) * L, L)] = zeros16
        return carry

    lax.fori_loop(0, CH * (D // L), _zero_r0, 0)
    for i in range(RPS // CH):
        pltpu.sync_copy(r0, agg_sh.at[pl.ds(sid * RPS + i * CH, CH)])
    _gather(tib_a.at[0, 0], rbufs[0], gsems[0])
    plsc.subcore_barrier()

    def _half(b, cur, nxt, isem_cur, isem_nxt):
        _blk_wait(nxt, isem_nxt)
        for m in range(NBLK):
            k = m % 3
            _gwait(rbufs[k], gsems[k])
            _scat(rbufs[k], cur.at[m, 1])
            if m < NBLK - 3:
                _gather(cur.at[m + 3, 0], rbufs[k], gsems[k])
            else:
                _gather(nxt.at[m + 3 - NBLK, 0], rbufs[k], gsems[k])
            _hist(cur, m)
        _blk_load(jnp.minimum(b + 2, NBLKS), cur, isem_cur)

    def _pair(i, carry):
        b = 2 * i
        _half(b, tib_a, tib_b, isem_a, isem_b)
        _half(b + 1, tib_b, tib_a, isem_b, isem_a)
        return carry

    lax.fori_loop(0, NBLKS // 2, _pair, 0)
    _blk_wait(tib_b, isem_b)
    for m in range(5):
        k = m % 3
        _gwait(rbufs[k], gsems[k])
        _scat(rbufs[k], tib_a.at[m, 1])
        if m + 3 < 5:
            _gather(tib_a.at[m + 3, 0], rbufs[k], gsems[k])
        _hist(tib_a, m)

    pltpu.sync_copy(hist_v, deg_hbm.at[wid])
    plsc.subcore_barrier()
    pltpu.sync_copy(agg_sh.at[pl.ds(sid * RPS, RPS)],
                    out_hbm.at[cid, pl.ds(sid * RPS, RPS)])


def _out_body(part_ref, deg_ref, h_ref, ws_ref, bs_ref, wn_ref, w2_ref,
              b2_ref, o_ref):
    agg = part_ref[0] + part_ref[1]
    degt = jnp.transpose(deg_ref[...])
    deg = jnp.maximum(jnp.sum(degt, axis=1, keepdims=True), 1.0)
    h_neigh = agg / deg
    h = h_ref[...]
    h2 = jnp.dot(h, ws_ref[...], preferred_element_type=jnp.float32)
    h2 = h2 + jnp.dot(h_neigh, wn_ref[...], preferred_element_type=jnp.float32)
    h2 = jnp.maximum(h2 + bs_ref[...], 0.0)
    o = jnp.dot(h2, w2_ref[...], preferred_element_type=jnp.float32)
    o_ref[...] = jnp.maximum(o + b2_ref[...], 0.0)


def _out(part, deg, h, ws, bs, wn, w2, b2):
    return pl.pallas_call(
        _out_body,
        grid=(NP // BN,),
        in_specs=[
            pl.BlockSpec((NC, BN, D), lambda i: (0, i, 0)),
            pl.BlockSpec((NW, BN), lambda i: (0, i)),
            pl.BlockSpec((BN, D), lambda i: (i, 0)),
            pl.BlockSpec((D, D), lambda i: (0, 0)),
            pl.BlockSpec((1, D), lambda i: (0, 0)),
            pl.BlockSpec((D, D), lambda i: (0, 0)),
            pl.BlockSpec((D, D), lambda i: (0, 0)),
            pl.BlockSpec((1, D), lambda i: (0, 0)),
        ],
        out_specs=pl.BlockSpec((BN, D), lambda i: (i, 0)),
        out_shape=jax.ShapeDtypeStruct((N, D), jnp.float32),
    )(part, deg, h, ws, bs, wn, w2, b2)


def kernel(x, edge_index, W_d1, b_d1, W_self, b_self, W_neigh, W_d2, b_d2):
    h = _d1(x, W_d1, b_d1.reshape(1, D))
    idx = jnp.transpose(edge_index.reshape(2, NW, NCHUNK, CH), (1, 2, 0, 3))
    idx = jnp.pad(idx, ((0, 0), (0, NCP - NCHUNK), (0, 0), (0, 0)))
    part, deg = _sc_agg(h, idx)
    return _out(part, deg, h, W_self,
                b_self.reshape(1, D), W_neigh, W_d2, b_d2.reshape(1, D))
